# baseline (speedup 1.0000x reference)
import functools

import jax
import jax.numpy as jnp
from jax import lax
from jax.experimental import pallas as pl
from jax.experimental.pallas import tpu as pltpu
from jax.experimental.pallas import tpu_sc as plsc

_LANES = 16


def _build_kernel(B, W, C):
    info = plsc.get_sparse_core_info()
    NW = info.num_cores * info.num_subcores
    assert C == _LANES
    assert B % NW == 0
    rows_per_tile = B // NW
    CH = 2
    assert rows_per_tile % CH == 0
    n_chunks = rows_per_tile // CH
    pos_unroll = 8
    assert W % pos_unroll == 0
    n_steps = W // pos_unroll

    mesh = plsc.VectorSubcoreMesh(core_axis_name="c", subcore_axis_name="s")

    @functools.partial(
        pl.kernel,
        out_type=jax.ShapeDtypeStruct((NW * _LANES,), jnp.float32),
        mesh=mesh,
        compiler_params=pltpu.CompilerParams(needs_layout_passes=False, use_tc_tiling_on_sc=True),
        scratch_types=[
            pltpu.VMEM((CH, W, C), jnp.float32),
            pltpu.VMEM((CH, W, C), jnp.float32),
            pltpu.VMEM((rows_per_tile * W + _LANES,), jnp.int32),
            pltpu.VMEM((_LANES,), jnp.float32),
            pltpu.SemaphoreType.DMA,
            pltpu.SemaphoreType.DMA,
        ],
    )
    def run(pred_hbm, tgt_hbm, out_hbm, pbuf0, pbuf1, tbuf, obuf, sem0, sem1):
        cid = lax.axis_index("c")
        sid = lax.axis_index("s")
        wid = sid * info.num_cores + cid
        row_base = wid * rows_per_tile

        bufs = [pbuf0, pbuf1]
        sems = [sem0, sem1]
        handles = {}
        handles[0] = pltpu.async_copy(
            pred_hbm.at[pl.ds(row_base, CH)], pbuf0, sem0)
        if n_chunks > 1:
            handles[1] = pltpu.async_copy(
                pred_hbm.at[pl.ds(row_base + CH, CH)], pbuf1, sem1)
        pltpu.sync_copy(tgt_hbm.at[pl.ds(row_base * W, rows_per_tile * W)],
                        tbuf.at[pl.ds(0, rows_per_tile * W)])

        iota = lax.iota(jnp.int32, _LANES)
        zeros_i = jnp.zeros((_LANES,), jnp.int32)
        obuf[...] = jnp.zeros((_LANES,), jnp.float32)

        for ci in range(n_chunks):
            handles[ci].wait()
            pbuf = bufs[ci % 2]

            def row_body(r, _, pbuf=pbuf, ci=ci):
                t_row = (ci * CH + r) * W

                def step_body(s, carry):
                    run_p, best_p, run_t, best_t = carry
                    w0 = s * pos_unroll
                    tv = tbuf[pl.ds(t_row + w0, _LANES)]
                    for j in range(pos_unroll):
                        x = pbuf[r, w0 + j, :]
                        m = jnp.max(x)
                        f = plsc.all_reduce_ffs(x == m)
                        mask_p = iota == f
                        run_p = jnp.where(mask_p, run_p + 1, zeros_i)
                        best_p = jnp.maximum(best_p, run_p)
                        mask_t = iota == tv[j]
                        run_t = jnp.where(mask_t, run_t + 1, zeros_i)
                        best_t = jnp.maximum(best_t, run_t)
                    return run_p, best_p, run_t, best_t

                _, best_p, _, best_t = lax.fori_loop(
                    0, n_steps, step_body,
                    (zeros_i, zeros_i, zeros_i, zeros_i))
                d = (best_p - best_t).astype(jnp.float32)
                obuf[...] = obuf[...] + d * d
                return 0

            lax.fori_loop(0, CH, row_body, 0)

            nxt = ci + 2
            if nxt < n_chunks:
                handles[nxt] = pltpu.async_copy(
                    pred_hbm.at[pl.ds(row_base + nxt * CH, CH)],
                    bufs[nxt % 2], sems[nxt % 2])

        pltpu.sync_copy(obuf, out_hbm.at[pl.ds(wid * _LANES, _LANES)])

    return run


def kernel(predictions, targets):
    B, W, C = predictions.shape
    run = _build_kernel(B, W, C)
    partials = run(predictions, targets.astype(jnp.int32).reshape(-1))
    return jnp.sum(partials) / jnp.float32(B * C)


# speedup vs baseline: 1.1229x; 1.1229x over previous
import functools

import jax
import jax.numpy as jnp
from jax import lax
from jax.experimental import pallas as pl
from jax.experimental.pallas import tpu as pltpu
from jax.experimental.pallas import tpu_sc as plsc

_LANES = 16


def _build_kernel(B, W, C):
    info = plsc.get_sparse_core_info()
    NW = info.num_cores * info.num_subcores
    assert C == _LANES
    assert B % NW == 0
    rows_per_tile = B // NW
    CH = 2
    assert rows_per_tile % (2 * CH) == 0
    n_chunks = rows_per_tile // CH
    n_pairs = n_chunks // 2
    FR = 8
    assert W % FR == 0
    n_steps = W // FR
    CWC = CH * W * C

    mesh = plsc.VectorSubcoreMesh(core_axis_name="c", subcore_axis_name="s")

    @functools.partial(
        pl.kernel,
        out_type=jax.ShapeDtypeStruct((NW * _LANES,), jnp.float32),
        mesh=mesh,
        compiler_params=pltpu.CompilerParams(needs_layout_passes=False),
        scratch_types=[
            pltpu.VMEM((CWC,), jnp.float32),
            pltpu.VMEM((CWC,), jnp.float32),
            pltpu.VMEM((rows_per_tile * W + _LANES,), jnp.int32),
            pltpu.VMEM((_LANES,), jnp.float32),
            pltpu.SemaphoreType.DMA,
            pltpu.SemaphoreType.DMA,
        ],
    )
    def run(pred_hbm, tgt_hbm, out_hbm, pbuf0, pbuf1, tbuf, obuf, sem0, sem1):
        cid = lax.axis_index("c")
        sid = lax.axis_index("s")
        wid = sid * info.num_cores + cid
        row_base = wid * rows_per_tile
        pred_base = row_base * W * C

        pltpu.async_copy(pred_hbm.at[pl.ds(pred_base, CWC)], pbuf0, sem0)
        pltpu.async_copy(pred_hbm.at[pl.ds(pred_base + CWC, CWC)],
                         pbuf1, sem1)
        pltpu.sync_copy(tgt_hbm.at[pl.ds(row_base * W, rows_per_tile * W)],
                        tbuf.at[pl.ds(0, rows_per_tile * W)])

        iota = lax.iota(jnp.int32, _LANES)
        zeros_i = jnp.zeros((_LANES,), jnp.uint32)

        def chunk(acc, chunk_row, pbuf):
            def step_body(s, carry):
                run_p = list(carry[0:CH])
                best_p = list(carry[CH:2 * CH])
                run_t = list(carry[2 * CH:3 * CH])
                best_t = list(carry[3 * CH:4 * CH])
                w0 = s * FR
                tvs = [tbuf[pl.ds((chunk_row + r) * W + w0, _LANES)]
                       for r in range(CH)]
                for j in range(FR):
                    for r in range(CH):
                        x = pbuf[pl.ds((r * W + w0 + j) * C, _LANES)]
                        m = jnp.max(x)
                        f = plsc.all_reduce_ffs(x == m)
                        mask_p = iota == f
                        run_p[r] = jnp.where(mask_p, run_p[r] + 1, zeros_i)
                        best_p[r] = jnp.maximum(best_p[r], run_p[r])
                        mask_t = iota == tvs[r][j]
                        run_t[r] = jnp.where(mask_t, run_t[r] + 1, zeros_i)
                        best_t[r] = jnp.maximum(best_t[r], run_t[r])
                return tuple(run_p + best_p + run_t + best_t)

            out = lax.fori_loop(0, n_steps, step_body,
                                tuple([zeros_i] * (4 * CH)))
            for r in range(CH):
                d = (out[CH + r].astype(jnp.int32)
                     - out[3 * CH + r].astype(jnp.int32)).astype(jnp.float32)
                acc = acc + d * d
            return acc

        def pair_body(i, acc):
            c0 = 2 * i

            pltpu.make_async_copy(
                pred_hbm.at[pl.ds(pred_base + c0 * CWC, CWC)],
                pbuf0, sem0).wait()
            acc = chunk(acc, c0 * CH, pbuf0)

            @pl.when(i < n_pairs - 1)
            def _():
                pltpu.async_copy(
                    pred_hbm.at[pl.ds(pred_base + (c0 + 2) * CWC, CWC)],
                    pbuf0, sem0)

            pltpu.make_async_copy(
                pred_hbm.at[pl.ds(pred_base + (c0 + 1) * CWC, CWC)],
                pbuf1, sem1).wait()
            acc = chunk(acc, (c0 + 1) * CH, pbuf1)

            @pl.when(i < n_pairs - 1)
            def _():
                pltpu.async_copy(
                    pred_hbm.at[pl.ds(pred_base + (c0 + 3) * CWC, CWC)],
                    pbuf1, sem1)

            return acc

        acc = lax.fori_loop(0, n_pairs, pair_body,
                            jnp.zeros((_LANES,), jnp.float32))

        obuf[...] = acc
        pltpu.sync_copy(obuf, out_hbm.at[pl.ds(wid * _LANES, _LANES)])

    return run


def kernel(predictions, targets):
    B, W, C = predictions.shape
    run = _build_kernel(B, W, C)
    partials = run(predictions.reshape(-1),
                   targets.astype(jnp.int32).reshape(-1))
    return jnp.sum(partials) / jnp.float32(B * C)


# speedup vs baseline: 5.3874x; 4.7977x over previous
import functools

import jax
import jax.numpy as jnp
from jax import lax
from jax.experimental import pallas as pl
from jax.experimental.pallas import tpu as pltpu
from jax.experimental.pallas import tpu_sc as plsc

_LANES = 16


def _build_kernel(B, W, C):
    info = plsc.get_sparse_core_info()
    NW = info.num_cores * info.num_subcores
    assert C == _LANES
    assert B % NW == 0
    rows_per_tile = B // NW
    n_groups = rows_per_tile // _LANES
    WF = 20
    assert W % WF == 0
    n_chunks = W // WF
    assert n_chunks >= 3
    NSTATE = 2 * C + 4

    mesh = plsc.VectorSubcoreMesh(core_axis_name="c", subcore_axis_name="s")

    @functools.partial(
        pl.kernel,
        out_type=jax.ShapeDtypeStruct((NW * _LANES,), jnp.float32),
        mesh=mesh,
        compiler_params=pltpu.CompilerParams(needs_layout_passes=False),
        scratch_types=[
            pltpu.VMEM((WF, C, rows_per_tile), jnp.float32),
            pltpu.VMEM((WF, C, rows_per_tile), jnp.float32),
            pltpu.VMEM((W, rows_per_tile), jnp.int32),
            pltpu.VMEM((n_groups * NSTATE * _LANES,), jnp.int32),
            pltpu.VMEM((_LANES,), jnp.float32),
            pltpu.SemaphoreType.DMA,
            pltpu.SemaphoreType.DMA,
            pltpu.SemaphoreType.DMA,
        ],
    )
    def run(pred_hbm, tgt_hbm, out_hbm, pbuf0, pbuf1, tbuf, st, obuf,
            psem0, psem1, tsem):
        cid = lax.axis_index("c")
        sid = lax.axis_index("s")
        wid = sid * info.num_cores + cid
        row_base = wid * rows_per_tile

        pbufs = [pbuf0, pbuf1]
        psems = [psem0, psem1]
        ph = {}

        def issue(ci):
            w0 = ci * WF
            ph[ci] = pltpu.async_copy(
                pred_hbm.at[pl.ds(w0, WF), :, pl.ds(row_base, rows_per_tile)],
                pbufs[ci % 2], psems[ci % 2])

        issue(0)
        issue(1)
        th = pltpu.async_copy(
            tgt_hbm.at[:, pl.ds(row_base, rows_per_tile)], tbuf, tsem)

        neg1 = jnp.full((_LANES,), -1, jnp.int32)
        zero = jnp.zeros((_LANES,), jnp.int32)
        uzero = jnp.zeros((_LANES,), jnp.uint32)
        uone = jnp.full((_LANES,), 1, jnp.uint32)
        consts = [jnp.full((_LANES,), c, jnp.int32) for c in range(C)]

        def frame(pbuf, ci, s, g, state):
            prev_p, run_p, prev_t, run_t = state[0:4]
            best_p = state[4:4 + C]
            best_t = state[4 + C:4 + 2 * C]
            g16 = g * _LANES

            vals = [pbuf[s, c, pl.ds(g16, _LANES)] for c in range(C)]
            idxs = list(consts)
            while len(vals) > 1:
                nv, ni = [], []
                for k in range(0, len(vals), 2):
                    ge = vals[k] >= vals[k + 1]
                    nv.append(jnp.maximum(vals[k], vals[k + 1]))
                    ni.append(jnp.where(ge, idxs[k], idxs[k + 1]))
                vals, idxs = nv, ni
            cls = idxs[0]

            run_p = jnp.where(cls == prev_p, run_p + 1, uone)
            prev_p = cls
            tv = tbuf[ci * WF + s, pl.ds(g16, _LANES)]
            run_t = jnp.where(tv == prev_t, run_t + 1, uone)
            prev_t = tv

            best_p = [jnp.maximum(best_p[c],
                                  jnp.where(cls == consts[c], run_p, uzero))
                      for c in range(C)]
            best_t = [jnp.maximum(best_t[c],
                                  jnp.where(tv == consts[c], run_t, uzero))
                      for c in range(C)]
            return [prev_p, run_p, prev_t, run_t] + best_p + best_t

        def group_chunk(pbuf, ci, g, acc):
            base = g * NSTATE * _LANES
            if ci == 0:
                state = [neg1, uzero, neg1, uzero] + [uzero] * (2 * C)
            else:
                raw = [st[pl.ds(base + k * _LANES, _LANES)]
                       for k in range(NSTATE)]
                state = ([raw[0], raw[1].astype(jnp.uint32), raw[2],
                          raw[3].astype(jnp.uint32)]
                         + [v.astype(jnp.uint32) for v in raw[4:]])

            def body(s, carry):
                return tuple(frame(pbuf, ci, s, g, list(carry)))

            state = list(lax.fori_loop(0, WF, body, tuple(state)))

            if ci == n_chunks - 1:
                best_p = state[4:4 + C]
                best_t = state[4 + C:4 + 2 * C]
                for c in range(C):
                    d = (best_p[c].astype(jnp.int32)
                         - best_t[c].astype(jnp.int32)).astype(jnp.float32)
                    acc = acc + d * d
            else:
                for k in range(NSTATE):
                    st[pl.ds(base + k * _LANES, _LANES)] = (
                        state[k].astype(jnp.int32))
            return acc

        acc = jnp.zeros((_LANES,), jnp.float32)
        th.wait()
        for ci in range(n_chunks):
            ph[ci].wait()

            def gbody(g, a, ci=ci, pbuf=pbufs[ci % 2]):
                return group_chunk(pbuf, ci, g, a)

            acc = lax.fori_loop(0, n_groups, gbody, acc)
            if ci + 2 < n_chunks:
                issue(ci + 2)

        obuf[...] = acc
        pltpu.sync_copy(obuf, out_hbm.at[pl.ds(wid * _LANES, _LANES)])

    return run


def kernel(predictions, targets):
    B, W, C = predictions.shape
    run = _build_kernel(B, W, C)
    pred_v = jnp.transpose(predictions, (1, 2, 0))
    tgt_v = jnp.transpose(targets.astype(jnp.int32), (1, 0))
    partials = run(pred_v, tgt_v)
    return jnp.sum(partials) / jnp.float32(B * C)
